# Initial kernel scaffold; baseline (speedup 1.0000x reference)
import jax
import jax.numpy as jnp
from jax.experimental import pallas as pl


def kernel(text_ids, text_embedding_weight):
    raise NotImplementedError("write your pallas kernel here")


# speedup vs baseline: 1.7304x; 1.7304x over previous
import functools

import jax
import jax.numpy as jnp
from jax import lax
from jax.experimental import pallas as pl
from jax.experimental.pallas import tpu as pltpu
from jax.experimental.pallas import tpu_sc as plsc

VOCAB = 32000
D = 4096
B_TOT = 4 * 8192
NC, NS = 2, 16
NW = NC * NS
BPW = B_TOT // NW
K = 16
NCHUNK = BPW // K


def _gather_kernel(ids_hbm, table_hbm, out_hbm, idx_v, rows_v, sem):
    wid = lax.axis_index("s") * NC + lax.axis_index("c")
    base = wid * BPW
    pltpu.sync_copy(ids_hbm.at[pl.ds(base, BPW)], idx_v)

    def body(g, carry):
        start = pl.multiple_of(g * K, K)
        pltpu.async_copy(
            table_hbm.at[idx_v.at[pl.ds(start, K)]], rows_v, sem
        ).wait()
        pltpu.sync_copy(rows_v, out_hbm.at[pl.ds(base + start, K)])
        return carry

    lax.fori_loop(0, NCHUNK, body, 0)


def kernel(text_ids, text_embedding_weight):
    ids = text_ids.reshape(-1).astype(jnp.int32)
    mesh = plsc.VectorSubcoreMesh(core_axis_name="c", subcore_axis_name="s")
    run = functools.partial(
        pl.kernel,
        mesh=mesh,
        out_type=jax.ShapeDtypeStruct((B_TOT, D), jnp.float32),
        scratch_types=[
            pltpu.VMEM((BPW,), jnp.int32),
            pltpu.VMEM((K, D), jnp.float32),
            pltpu.SemaphoreType.DMA,
        ],
    )(_gather_kernel)
    out = run(ids, text_embedding_weight)
    return out.reshape(text_ids.shape + (D,))


# speedup vs baseline: 1.8652x; 1.0779x over previous
import functools

import jax
import jax.numpy as jnp
from jax import lax
from jax.experimental import pallas as pl
from jax.experimental.pallas import tpu as pltpu
from jax.experimental.pallas import tpu_sc as plsc

VOCAB = 32000
D = 4096
B_TOT = 4 * 8192
NC, NS = 2, 16
NW = NC * NS
BPW = B_TOT // NW
K = 8
NB = 2
NCHUNK = BPW // K


def _gather_kernel(ids_hbm, table_hbm, out_hbm, idx_v, rows0, rows1,
                   gsem0, gsem1, osem0, osem1):
    rows = (rows0, rows1)
    gsem = (gsem0, gsem1)
    osem = (osem0, osem1)
    wid = lax.axis_index("s") * NC + lax.axis_index("c")
    base = wid * BPW
    pltpu.sync_copy(ids_hbm.at[pl.ds(base, BPW)], idx_v)

    def gather(g, b):
        start = pl.multiple_of(g * K, K)
        pltpu.async_copy(
            table_hbm.at[idx_v.at[pl.ds(start, K)]], rows[b], gsem[b]
        )

    def put(g, b):
        start = pl.multiple_of(g * K, K)
        pltpu.async_copy(rows[b], out_hbm.at[pl.ds(base + start, K)], osem[b])

    for b in range(NB):
        gather(b, b)

    def body(i, carry):
        for b in range(NB):
            g = i * NB + b
            pltpu.make_async_copy(rows[b], out_hbm.at[pl.ds(0, K)],
                                  gsem[b]).wait()
            put(g, b)

            @pl.when(g + NB < NCHUNK)
            def _():
                pltpu.make_async_copy(rows[b], out_hbm.at[pl.ds(0, K)],
                                      osem[b]).wait()
                gather(g + NB, b)

        return carry

    lax.fori_loop(0, NCHUNK // NB, body, 0)

    for b in range(NB):
        pltpu.make_async_copy(rows[b], out_hbm.at[pl.ds(0, K)], osem[b]).wait()


def kernel(text_ids, text_embedding_weight):
    ids = text_ids.reshape(-1).astype(jnp.int32)
    mesh = plsc.VectorSubcoreMesh(core_axis_name="c", subcore_axis_name="s")
    run = functools.partial(
        pl.kernel,
        mesh=mesh,
        out_type=jax.ShapeDtypeStruct((B_TOT, D), jnp.float32),
        scratch_types=[
            pltpu.VMEM((BPW,), jnp.int32),
            pltpu.VMEM((K, D), jnp.float32),
            pltpu.VMEM((K, D), jnp.float32),
            pltpu.SemaphoreType.DMA,
            pltpu.SemaphoreType.DMA,
            pltpu.SemaphoreType.DMA,
            pltpu.SemaphoreType.DMA,
        ],
    )(_gather_kernel)
    out = run(ids, text_embedding_weight)
    return out.reshape(text_ids.shape + (D,))


# speedup vs baseline: 1.8760x; 1.0058x over previous
import functools

import jax
import jax.numpy as jnp
from jax import lax
from jax.experimental import pallas as pl
from jax.experimental.pallas import tpu as pltpu
from jax.experimental.pallas import tpu_sc as plsc

VOCAB = 32000
D = 4096
B_TOT = 4 * 8192
NC, NS = 2, 16
NW = NC * NS
BPW = B_TOT // NW
K = 8
NB = 3
NCHUNK = BPW // K
NSTEADY = (NCHUNK // NB) * NB


def _gather_kernel(ids_hbm, table_hbm, out_hbm, idx_v, rows0, rows1, rows2,
                   gsem0, gsem1, gsem2, osem0, osem1, osem2):
    rows = (rows0, rows1, rows2)
    gsem = (gsem0, gsem1, gsem2)
    osem = (osem0, osem1, osem2)
    wid = lax.axis_index("s") * NC + lax.axis_index("c")
    base = wid * BPW
    pltpu.sync_copy(ids_hbm.at[pl.ds(base, BPW)], idx_v)

    def gather(g, b):
        start = pl.multiple_of(g * K, K)
        pltpu.async_copy(
            table_hbm.at[idx_v.at[pl.ds(start, K)]], rows[b], gsem[b]
        )

    def put(g, b):
        start = pl.multiple_of(g * K, K)
        pltpu.async_copy(rows[b], out_hbm.at[pl.ds(base + start, K)], osem[b])

    def wait_gather(b):
        pltpu.make_async_copy(rows[b], out_hbm.at[pl.ds(0, K)], gsem[b]).wait()

    def wait_put(b):
        pltpu.make_async_copy(rows[b], out_hbm.at[pl.ds(0, K)], osem[b]).wait()

    for b in range(NB):
        gather(b, b)

    def body(i, carry):
        for b in range(NB):
            c = i * NB + b
            wait_gather(b)
            put(c, b)
            d = (b + NB - 1) % NB

            @pl.when((c >= 1) & (c + 2 < NCHUNK))
            def _():
                wait_put(d)
                gather(c + 2, d)

        return carry

    lax.fori_loop(0, NSTEADY // NB, body, 0)

    for c in range(NSTEADY, NCHUNK):
        b = c % NB
        wait_gather(b)
        put(c, b)

    for c in range(NCHUNK - NB, NCHUNK):
        wait_put(c % NB)


def kernel(text_ids, text_embedding_weight):
    ids = text_ids.reshape(-1).astype(jnp.int32)
    mesh = plsc.VectorSubcoreMesh(core_axis_name="c", subcore_axis_name="s")
    run = functools.partial(
        pl.kernel,
        mesh=mesh,
        out_type=jax.ShapeDtypeStruct((B_TOT, D), jnp.float32),
        scratch_types=[
            pltpu.VMEM((BPW,), jnp.int32),
            pltpu.VMEM((K, D), jnp.float32),
            pltpu.VMEM((K, D), jnp.float32),
            pltpu.VMEM((K, D), jnp.float32),
            pltpu.SemaphoreType.DMA,
            pltpu.SemaphoreType.DMA,
            pltpu.SemaphoreType.DMA,
            pltpu.SemaphoreType.DMA,
            pltpu.SemaphoreType.DMA,
            pltpu.SemaphoreType.DMA,
        ],
    )(_gather_kernel)
    out = run(ids, text_embedding_weight)
    return out.reshape(text_ids.shape + (D,))
